# Initial kernel scaffold; baseline (speedup 1.0000x reference)
import jax
import jax.numpy as jnp
from jax.experimental import pallas as pl


def kernel(states_action, states_graph_ids, states_first, states_second, sates_num_edges, W_msg, b_msg, w_gcn, b_gcn, W_r1, b_r1, W_r2, b_r2, W_r3, b_r3):
    raise NotImplementedError("write your pallas kernel here")


# speedup vs baseline: 2.1822x; 2.1822x over previous
import functools

import jax
import jax.numpy as jnp
from jax import lax
from jax.experimental import pallas as pl
from jax.experimental.pallas import tpu as pltpu
from jax.experimental.pallas import tpu_sc as plsc

N = 10000
E = 320000
D = 128
G = 256
RU = 256
T = 4

NC = 2
NS = 16
NW = NC * NS
PW = E // NW
CH = 80
NCHUNK = PW // CH
OWN = 624
TAIL = N - NS * OWN
ZR = 16

_SELU_SCALE = 1.0507009873554805
_SELU_ALPHA = 1.6732632423543772


def _expm1(x):
    u = jnp.exp(x)
    um1 = u - 1.0
    r = um1 * x / jnp.log(u)
    r = jnp.where(u == 1.0, x, r)
    r = jnp.where(um1 == -1.0, -1.0, r)
    return r


def _selu(x):
    xm = jnp.minimum(x, 0.0)
    return _SELU_SCALE * jnp.where(x > 0, x, _SELU_ALPHA * _expm1(xm))


_BN = 2000


def _ab_body_pair(p0_ref, p1_ref, w1_ref, w2_ref, b_ref, a_ref, b_out_ref):
    ls = p0_ref[...] + p1_ref[...]
    a_ref[...] = jnp.dot(ls, w1_ref[...], preferred_element_type=jnp.float32)
    b_out_ref[...] = (
        jnp.dot(ls, w2_ref[...], preferred_element_type=jnp.float32) + b_ref[...]
    )


def _ab_body_single(p0_ref, w1_ref, w2_ref, b_ref, a_ref, b_out_ref):
    ls = p0_ref[...]
    a_ref[...] = jnp.dot(ls, w1_ref[...], preferred_element_type=jnp.float32)
    b_out_ref[...] = (
        jnp.dot(ls, w2_ref[...], preferred_element_type=jnp.float32) + b_ref[...]
    )


def _make_ab(n_in):
    body = _ab_body_single if n_in == 1 else _ab_body_pair
    state_spec = pl.BlockSpec((_BN, D), lambda i: (i, 0))
    w_spec = pl.BlockSpec((D, D), lambda i: (0, 0))
    b_spec = pl.BlockSpec((1, D), lambda i: (0, 0))
    return pl.pallas_call(
        body,
        grid=(N // _BN,),
        in_specs=[state_spec] * n_in + [w_spec, w_spec, b_spec],
        out_specs=[state_spec, state_spec],
        out_shape=[
            jax.ShapeDtypeStruct((N, D), jnp.float32),
            jax.ShapeDtypeStruct((N, D), jnp.float32),
        ],
    )


def _ab_single(ls, w1, w2, b):
    return _make_ab(1)(ls, w1, w2, b.reshape(1, D))


def _ab_pair(p0, p1, w1, w2, b):
    return _make_ab(2)(p0, p1, w1, w2, b.reshape(1, D))


def _gather_body(a_hbm, b_hbm, f_hbm, s_hbm, out_hbm,
                 idx1, idx2, ra, rb, sem1, sem2):
    c = lax.axis_index("c")
    s = lax.axis_index("s")
    wid = c * NS + s
    base0 = wid * PW

    def chunk(i, carry):
        base = base0 + i * CH
        pltpu.sync_copy(f_hbm.at[pl.ds(base, CH)], idx1)
        pltpu.sync_copy(s_hbm.at[pl.ds(base, CH)], idx2)
        cp1 = pltpu.async_copy(a_hbm.at[idx1], ra, sem1)
        cp2 = pltpu.async_copy(b_hbm.at[idx2], rb, sem2)
        cp1.wait()
        cp2.wait()

        def row(r, carry2):
            for j in range(D // 16):
                sl = pl.ds(j * 16, 16)
                ra[r, sl] = ra[r, sl] + rb[r, sl]
            return carry2

        lax.fori_loop(0, CH, row, 0, unroll=2)
        pltpu.sync_copy(ra, out_hbm.at[pl.ds(base, CH)])
        return carry

    lax.fori_loop(0, NCHUNK, chunk, 0)


def _sc_gather(a, b, first, second):
    mesh = plsc.VectorSubcoreMesh(core_axis_name="c", subcore_axis_name="s")
    fn = pl.kernel(
        _gather_body,
        out_type=jax.ShapeDtypeStruct((E, D), jnp.float32),
        mesh=mesh,
        scratch_types=[
            pltpu.VMEM((CH,), jnp.int32),
            pltpu.VMEM((CH,), jnp.int32),
            pltpu.VMEM((CH, D), jnp.float32),
            pltpu.VMEM((CH, D), jnp.float32),
            pltpu.SemaphoreType.DMA,
            pltpu.SemaphoreType.DMA,
        ],
    )
    return fn(a, b, first, second)


_BE = 2000


def _edge_body(x_ref, w_ref, b_ref, o_ref):
    sx = _selu(x_ref[...])
    h = jnp.dot(sx, w_ref[...], preferred_element_type=jnp.float32) + b_ref[...]
    o_ref[...] = jnp.maximum(h, 0.0)


def _tc_edge(x, w, b):
    return pl.pallas_call(
        _edge_body,
        grid=(E // _BE,),
        in_specs=[
            pl.BlockSpec((_BE, D), lambda i: (i, 0)),
            pl.BlockSpec((D, D), lambda i: (0, 0)),
            pl.BlockSpec((1, D), lambda i: (0, 0)),
        ],
        out_specs=pl.BlockSpec((_BE, D), lambda i: (i, 0)),
        out_shape=jax.ShapeDtypeStruct((E, D), jnp.float32),
    )(x, w, b.reshape(1, D))


def _scatter_body(h_hbm, s_hbm, out_hbm, acc, idx, rows, zbuf):
    c = lax.axis_index("c")
    s = lax.axis_index("s")

    for r in range(ZR):
        for j in range(D // 16):
            zbuf[r, pl.ds(j * 16, 16)] = jnp.zeros((16,), jnp.float32)

    def zero_chunk(i, carry):
        pltpu.sync_copy(zbuf, acc.at[pl.ds(s * OWN + i * ZR, ZR)])
        return carry

    lax.fori_loop(0, OWN // ZR, zero_chunk, 0)

    @pl.when(s == NS - 1)
    def _():
        pltpu.sync_copy(zbuf, acc.at[pl.ds(NS * OWN, TAIL)])

    plsc.subcore_barrier()

    base0 = (c * NS + s) * PW

    def chunk(i, carry):
        base = base0 + i * CH
        pltpu.sync_copy(s_hbm.at[pl.ds(base, CH)], idx)
        pltpu.sync_copy(h_hbm.at[pl.ds(base, CH)], rows)
        pltpu.sync_copy(rows, acc.at[idx], add=True)
        return carry

    lax.fori_loop(0, NCHUNK, chunk, 0)
    plsc.subcore_barrier()

    pltpu.sync_copy(
        acc.at[pl.ds(s * OWN, OWN)],
        out_hbm.at[c, pl.ds(s * OWN, OWN)],
    )

    @pl.when(s == NS - 1)
    def _():
        pltpu.sync_copy(
            acc.at[pl.ds(NS * OWN, TAIL)],
            out_hbm.at[c, pl.ds(NS * OWN, TAIL)],
        )


def _sc_scatter(h, second):
    mesh = plsc.VectorSubcoreMesh(core_axis_name="c", subcore_axis_name="s")
    fn = pl.kernel(
        _scatter_body,
        out_type=jax.ShapeDtypeStruct((NC, N, D), jnp.float32),
        mesh=mesh,
        scratch_types=[
            pltpu.VMEM_SHARED((N, D), jnp.float32),
            pltpu.VMEM((CH,), jnp.int32),
            pltpu.VMEM((CH, D), jnp.float32),
            pltpu.VMEM((ZR, D), jnp.float32),
        ],
    )
    return fn(h, second)


_BR = 1000


def _readout_body(gid_ref, p0_ref, p1_ref, w1_ref, b1_ref, w2_ref, b2_ref,
                  w3_ref, b3_ref, out_ref, acc_ref):
    i = pl.program_id(0)

    @pl.when(i == 0)
    def _():
        acc_ref[...] = jnp.zeros_like(acc_ref)

    ls = p0_ref[...] + p1_ref[...]
    ids = gid_ref[0]
    onehot = (lax.broadcasted_iota(jnp.int32, (G, _BR), 0) == ids).astype(
        jnp.float32
    )
    acc_ref[...] += jnp.dot(onehot, ls, preferred_element_type=jnp.float32,
                            precision=lax.Precision.HIGHEST)

    @pl.when(i == N // _BR - 1)
    def _():
        r = _selu(
            jnp.dot(acc_ref[...], w1_ref[...], preferred_element_type=jnp.float32)
            + b1_ref[...]
        )
        r = _selu(
            jnp.dot(r, w2_ref[...], preferred_element_type=jnp.float32)
            + b2_ref[...]
        )
        out_ref[...] = (
            jnp.dot(r, w3_ref[...], preferred_element_type=jnp.float32)
            + b3_ref[...]
        )


def _tc_readout(gids, p0, p1, w1, b1, w2, b2, w3, b3):
    return pl.pallas_call(
        _readout_body,
        grid=(N // _BR,),
        in_specs=[
            pl.BlockSpec((1, 1, _BR), lambda i: (i, 0, 0)),
            pl.BlockSpec((_BR, D), lambda i: (i, 0)),
            pl.BlockSpec((_BR, D), lambda i: (i, 0)),
            pl.BlockSpec((D, RU), lambda i: (0, 0)),
            pl.BlockSpec((1, RU), lambda i: (0, 0)),
            pl.BlockSpec((RU, RU), lambda i: (0, 0)),
            pl.BlockSpec((1, RU), lambda i: (0, 0)),
            pl.BlockSpec((RU, 1), lambda i: (0, 0)),
            pl.BlockSpec((1, 1), lambda i: (0, 0)),
        ],
        out_specs=pl.BlockSpec((G, 1), lambda i: (0, 0)),
        out_shape=jax.ShapeDtypeStruct((G, 1), jnp.float32),
        scratch_shapes=[pltpu.VMEM((G, D), jnp.float32)],
    )(
        gids.reshape(N // _BR, 1, _BR),
        p0,
        p1,
        w1,
        b1.reshape(1, RU),
        w2,
        b2.reshape(1, RU),
        w3,
        b3.reshape(1, 1),
    )


def kernel(states_action, states_graph_ids, states_first, states_second,
           sates_num_edges, W_msg, b_msg, w_gcn, b_gcn,
           W_r1, b_r1, W_r2, b_r2, W_r3, b_r3):
    w1 = W_msg[:D]
    w2 = W_msg[D:]
    a, b = _ab_single(states_action, w1, w2, b_msg)
    p = None
    for t in range(T):
        x = _sc_gather(a, b, states_first, states_second)
        h = _tc_edge(x, w_gcn, b_gcn)
        p = _sc_scatter(h, states_second)
        if t < T - 1:
            a, b = _ab_pair(p[0], p[1], w1, w2, b_msg)
    return _tc_readout(states_graph_ids, p[0], p[1],
                       W_r1, b_r1, W_r2, b_r2, W_r3, b_r3)


# speedup vs baseline: 3.2343x; 1.4821x over previous
import functools

import jax
import jax.numpy as jnp
from jax import lax
from jax.experimental import pallas as pl
from jax.experimental.pallas import tpu as pltpu
from jax.experimental.pallas import tpu_sc as plsc

N = 10000
E = 320000
D = 128
G = 256
RU = 256
T = 4

NC = 2
NS = 16
NW = NC * NS
PW = E // NW
CH = 40
NCHUNK = PW // CH
NPAIR = NCHUNK // 2
WIN = 264
OWN = 624
TAIL = N - NS * OWN
ZR = 16

_SELU_SCALE = 1.0507009873554805
_SELU_ALPHA = 1.6732632423543772


def _expm1(x):
    u = jnp.exp(x)
    um1 = u - 1.0
    r = um1 * x / jnp.log(u)
    r = jnp.where(u == 1.0, x, r)
    r = jnp.where(um1 == -1.0, -1.0, r)
    return r


def _selu(x):
    xm = jnp.minimum(x, 0.0)
    return _SELU_SCALE * jnp.where(x > 0, x, _SELU_ALPHA * _expm1(xm))


_BN = 2000


def _ab_body_pair(p0_ref, p1_ref, w1_ref, w2_ref, b_ref, a_ref, b_out_ref):
    ls = p0_ref[...] + p1_ref[...]
    a_ref[...] = jnp.dot(ls, w1_ref[...], preferred_element_type=jnp.float32)
    b_out_ref[...] = (
        jnp.dot(ls, w2_ref[...], preferred_element_type=jnp.float32) + b_ref[...]
    )


def _ab_body_single(p0_ref, w1_ref, w2_ref, b_ref, a_ref, b_out_ref):
    ls = p0_ref[...]
    a_ref[...] = jnp.dot(ls, w1_ref[...], preferred_element_type=jnp.float32)
    b_out_ref[...] = (
        jnp.dot(ls, w2_ref[...], preferred_element_type=jnp.float32) + b_ref[...]
    )


def _make_ab(n_in):
    body = _ab_body_single if n_in == 1 else _ab_body_pair
    state_spec = pl.BlockSpec((_BN, D), lambda i: (i, 0))
    w_spec = pl.BlockSpec((D, D), lambda i: (0, 0))
    b_spec = pl.BlockSpec((1, D), lambda i: (0, 0))
    return pl.pallas_call(
        body,
        grid=(N // _BN,),
        in_specs=[state_spec] * n_in + [w_spec, w_spec, b_spec],
        out_specs=[state_spec, state_spec],
        out_shape=[
            jax.ShapeDtypeStruct((N, D), jnp.float32),
            jax.ShapeDtypeStruct((N, D), jnp.float32),
        ],
    )


def _ab_single(ls, w1, w2, b):
    return _make_ab(1)(ls, w1, w2, b.reshape(1, D))


def _ab_pair(p0, p1, w1, w2, b):
    return _make_ab(2)(p0, p1, w1, w2, b.reshape(1, D))


def _add_rows(ra, rb, ro):
    def row(r, carry):
        for j in range(D // 16):
            sl = pl.ds(j * 16, 16)
            ro[r, sl] = ra[r, sl] + rb[r, sl]
        return carry

    lax.fori_loop(0, CH, row, 0, unroll=4)


def _gather_body(a_hbm, b_hbm, f_hbm, s_hbm, out_hbm,
                 fidx, sidx,
                 ra0, rb0, ro0, ra1, rb1, ro1,
                 sga0, sgb0, sga1, sgb1, sso0, sso1):
    c = lax.axis_index("c")
    s = lax.axis_index("s")
    wid = c * NS + s
    row0 = wid * NCHUNK
    aligned0 = (row0 // 8) * 8
    off = row0 - aligned0

    cpf = pltpu.async_copy(f_hbm.at[pl.ds(aligned0, WIN)], fidx, sga0)
    cps = pltpu.async_copy(s_hbm.at[pl.ds(aligned0, WIN)], sidx, sgb0)
    cpf.wait()
    cps.wait()

    def issue(j, ra, rb, sa, sb):
        pltpu.async_copy(a_hbm.at[fidx.at[j + off]], ra, sa)
        pltpu.async_copy(b_hbm.at[sidx.at[j + off]], rb, sb)

    def wait_gather(ra, rb, sa, sb):
        pltpu.make_async_copy(a_hbm.at[pl.ds(0, CH)], ra, sa).wait()
        pltpu.make_async_copy(b_hbm.at[pl.ds(0, CH)], rb, sb).wait()

    def wait_store(ro, so):
        pltpu.make_async_copy(ro, out_hbm.at[pl.ds(0, CH)], so).wait()

    issue(0, ra0, rb0, sga0, sgb0)

    def pair(g, carry):
        j0 = 2 * g
        issue(j0 + 1, ra1, rb1, sga1, sgb1)
        wait_gather(ra0, rb0, sga0, sgb0)

        @pl.when(g > 0)
        def _():
            wait_store(ro0, sso0)

        _add_rows(ra0, rb0, ro0)
        pltpu.async_copy(ro0, out_hbm.at[pl.ds((row0 + j0) * CH, CH)], sso0)

        @pl.when(g < NPAIR - 1)
        def _():
            issue(j0 + 2, ra0, rb0, sga0, sgb0)

        wait_gather(ra1, rb1, sga1, sgb1)

        @pl.when(g > 0)
        def _():
            wait_store(ro1, sso1)

        _add_rows(ra1, rb1, ro1)
        pltpu.async_copy(ro1, out_hbm.at[pl.ds((row0 + j0 + 1) * CH, CH)], sso1)
        return carry

    lax.fori_loop(0, NPAIR, pair, 0)
    wait_store(ro0, sso0)
    wait_store(ro1, sso1)


def _sc_gather(a, b, f2d, s2d):
    mesh = plsc.VectorSubcoreMesh(core_axis_name="c", subcore_axis_name="s")
    fn = pl.kernel(
        _gather_body,
        out_type=jax.ShapeDtypeStruct((E, D), jnp.float32),
        mesh=mesh,
        scratch_types=[
            pltpu.VMEM((WIN, CH), jnp.int32),
            pltpu.VMEM((WIN, CH), jnp.int32),
            pltpu.VMEM((CH, D), jnp.float32),
            pltpu.VMEM((CH, D), jnp.float32),
            pltpu.VMEM((CH, D), jnp.float32),
            pltpu.VMEM((CH, D), jnp.float32),
            pltpu.VMEM((CH, D), jnp.float32),
            pltpu.VMEM((CH, D), jnp.float32),
            pltpu.SemaphoreType.DMA,
            pltpu.SemaphoreType.DMA,
            pltpu.SemaphoreType.DMA,
            pltpu.SemaphoreType.DMA,
            pltpu.SemaphoreType.DMA,
            pltpu.SemaphoreType.DMA,
        ],
    )
    return fn(a, b, f2d, s2d)


_BE = 2000


def _edge_body(x_ref, w_ref, b_ref, o_ref):
    sx = _selu(x_ref[...])
    h = jnp.dot(sx, w_ref[...], preferred_element_type=jnp.float32) + b_ref[...]
    o_ref[...] = jnp.maximum(h, 0.0)


def _tc_edge(x, w, b):
    return pl.pallas_call(
        _edge_body,
        grid=(E // _BE,),
        in_specs=[
            pl.BlockSpec((_BE, D), lambda i: (i, 0)),
            pl.BlockSpec((D, D), lambda i: (0, 0)),
            pl.BlockSpec((1, D), lambda i: (0, 0)),
        ],
        out_specs=pl.BlockSpec((_BE, D), lambda i: (i, 0)),
        out_shape=jax.ShapeDtypeStruct((E, D), jnp.float32),
    )(x, w, b.reshape(1, D))


def _scatter_body(h_hbm, s_hbm, out_hbm, acc, sidx, r0, r1, zbuf,
                  sl0, sl1, sidle):
    c = lax.axis_index("c")
    s = lax.axis_index("s")
    row0 = (c * NS + s) * NCHUNK
    aligned0 = (row0 // 8) * 8
    off = row0 - aligned0

    cpi = pltpu.async_copy(s_hbm.at[pl.ds(aligned0, WIN)], sidx, sidle)

    for r in range(ZR):
        for j in range(D // 16):
            zbuf[r, pl.ds(j * 16, 16)] = jnp.zeros((16,), jnp.float32)

    def zero_chunk(i, carry):
        pltpu.sync_copy(zbuf, acc.at[pl.ds(s * OWN + i * ZR, ZR)])
        return carry

    lax.fori_loop(0, OWN // ZR, zero_chunk, 0)

    @pl.when(s == NS - 1)
    def _():
        pltpu.sync_copy(zbuf, acc.at[pl.ds(NS * OWN, TAIL)])

    cpi.wait()
    plsc.subcore_barrier()

    def issue(j, rbuf, sem):
        pltpu.async_copy(h_hbm.at[pl.ds((row0 + j) * CH, CH)], rbuf, sem)

    def wait_load(rbuf, sem):
        pltpu.make_async_copy(h_hbm.at[pl.ds(0, CH)], rbuf, sem).wait()

    issue(0, r0, sl0)

    def pair(g, carry):
        j0 = 2 * g
        issue(j0 + 1, r1, sl1)
        wait_load(r0, sl0)
        pltpu.sync_copy(r0, acc.at[sidx.at[j0 + off]], add=True)

        @pl.when(g < NPAIR - 1)
        def _():
            issue(j0 + 2, r0, sl0)

        wait_load(r1, sl1)
        pltpu.sync_copy(r1, acc.at[sidx.at[j0 + 1 + off]], add=True)
        return carry

    lax.fori_loop(0, NPAIR, pair, 0)
    plsc.subcore_barrier()

    pltpu.sync_copy(
        acc.at[pl.ds(s * OWN, OWN)],
        out_hbm.at[c, pl.ds(s * OWN, OWN)],
    )

    @pl.when(s == NS - 1)
    def _():
        pltpu.sync_copy(
            acc.at[pl.ds(NS * OWN, TAIL)],
            out_hbm.at[c, pl.ds(NS * OWN, TAIL)],
        )


def _sc_scatter(h, second):
    mesh = plsc.VectorSubcoreMesh(core_axis_name="c", subcore_axis_name="s")
    fn = pl.kernel(
        _scatter_body,
        out_type=jax.ShapeDtypeStruct((NC, N, D), jnp.float32),
        mesh=mesh,
        scratch_types=[
            pltpu.VMEM_SHARED((N, D), jnp.float32),
            pltpu.VMEM((WIN, CH), jnp.int32),
            pltpu.VMEM((CH, D), jnp.float32),
            pltpu.VMEM((CH, D), jnp.float32),
            pltpu.VMEM((ZR, D), jnp.float32),
            pltpu.SemaphoreType.DMA,
            pltpu.SemaphoreType.DMA,
            pltpu.SemaphoreType.DMA,
        ],
    )
    return fn(h, second)


_BR = 1000


def _readout_body(gid_ref, p0_ref, p1_ref, w1_ref, b1_ref, w2_ref, b2_ref,
                  w3_ref, b3_ref, out_ref, acc_ref):
    i = pl.program_id(0)

    @pl.when(i == 0)
    def _():
        acc_ref[...] = jnp.zeros_like(acc_ref)

    ls = p0_ref[...] + p1_ref[...]
    ids = gid_ref[0]
    onehot = (lax.broadcasted_iota(jnp.int32, (G, _BR), 0) == ids).astype(
        jnp.float32
    )
    acc_ref[...] += jnp.dot(onehot, ls, preferred_element_type=jnp.float32,
                            precision=lax.Precision.HIGHEST)

    @pl.when(i == N // _BR - 1)
    def _():
        r = _selu(
            jnp.dot(acc_ref[...], w1_ref[...], preferred_element_type=jnp.float32)
            + b1_ref[...]
        )
        r = _selu(
            jnp.dot(r, w2_ref[...], preferred_element_type=jnp.float32)
            + b2_ref[...]
        )
        out_ref[...] = (
            jnp.dot(r, w3_ref[...], preferred_element_type=jnp.float32)
            + b3_ref[...]
        )


def _tc_readout(gids, p0, p1, w1, b1, w2, b2, w3, b3):
    return pl.pallas_call(
        _readout_body,
        grid=(N // _BR,),
        in_specs=[
            pl.BlockSpec((1, 1, _BR), lambda i: (i, 0, 0)),
            pl.BlockSpec((_BR, D), lambda i: (i, 0)),
            pl.BlockSpec((_BR, D), lambda i: (i, 0)),
            pl.BlockSpec((D, RU), lambda i: (0, 0)),
            pl.BlockSpec((1, RU), lambda i: (0, 0)),
            pl.BlockSpec((RU, RU), lambda i: (0, 0)),
            pl.BlockSpec((1, RU), lambda i: (0, 0)),
            pl.BlockSpec((RU, 1), lambda i: (0, 0)),
            pl.BlockSpec((1, 1), lambda i: (0, 0)),
        ],
        out_specs=pl.BlockSpec((G, 1), lambda i: (0, 0)),
        out_shape=jax.ShapeDtypeStruct((G, 1), jnp.float32),
        scratch_shapes=[pltpu.VMEM((G, D), jnp.float32)],
    )(
        gids.reshape(N // _BR, 1, _BR),
        p0,
        p1,
        w1,
        b1.reshape(1, RU),
        w2,
        b2.reshape(1, RU),
        w3,
        b3.reshape(1, 1),
    )


def kernel(states_action, states_graph_ids, states_first, states_second,
           sates_num_edges, W_msg, b_msg, w_gcn, b_gcn,
           W_r1, b_r1, W_r2, b_r2, W_r3, b_r3):
    w1 = W_msg[:D]
    w2 = W_msg[D:]
    f2d = jnp.pad(states_first.reshape(E // CH, CH), ((0, 8), (0, 0)))
    s2d = jnp.pad(states_second.reshape(E // CH, CH), ((0, 8), (0, 0)))
    a, b = _ab_single(states_action, w1, w2, b_msg)
    p = None
    for t in range(T):
        x = _sc_gather(a, b, f2d, s2d)
        h = _tc_edge(x, w_gcn, b_gcn)
        p = _sc_scatter(h, s2d)
        if t < T - 1:
            a, b = _ab_pair(p[0], p[1], w1, w2, b_msg)
    return _tc_readout(states_graph_ids, p[0], p[1],
                       W_r1, b_r1, W_r2, b_r2, W_r3, b_r3)
